# Initial kernel scaffold; baseline (speedup 1.0000x reference)
import jax
import jax.numpy as jnp
from jax.experimental import pallas as pl


def kernel(features, edge_index, edge_index2, edge_weight, edge_weight2, ln1_W, ln1_b, c1_W, c1_b, c2_W, c2_b, bn1_g, bn1_b, ln2_W, ln2_b, c3_W, c3_b, c4_W, c4_b, bn2_g, bn2_b, conv_W, conv_b):
    raise NotImplementedError("write your pallas kernel here")


# speedup vs baseline: 2.8924x; 2.8924x over previous
import functools

import jax
import jax.numpy as jnp
from jax import lax
from jax.experimental import pallas as pl
from jax.experimental.pallas import tpu as pltpu
from jax.experimental.pallas import tpu_sc as plsc

N = 10000
E = 320000
F_IN = 128
H = 128
C = 40
CP = 48

NC = 2
NS = 16
NW = NC * NS
EPW = E // NW
K = 80
NCHUNK = EPW // K
NP = 10240
RPT = NP // NS
ZR = 128


def _make_sc_conv(D):
    nsl = D // 16
    mesh = plsc.VectorSubcoreMesh(core_axis_name="c", subcore_axis_name="s")

    @functools.partial(
        pl.kernel,
        out_type=jax.ShapeDtypeStruct((NC * NP, D), jnp.float32),
        mesh=mesh,
        scratch_types=[
            pltpu.VMEM((K,), jnp.int32),
            pltpu.VMEM((K,), jnp.int32),
            pltpu.VMEM((K, 16), jnp.float32),
            pltpu.VMEM((K, D), jnp.float32),
            pltpu.VMEM((ZR, D), jnp.float32),
            pltpu.VMEM_SHARED((NP, D), jnp.float32),
            pltpu.SemaphoreType.DMA,
        ],
        compiler_params=pltpu.CompilerParams(use_tc_tiling_on_sc=False),
    )
    def conv(h_hbm, src_hbm, dst_hbm, wexp_hbm, out_hbm,
             src_v, dst_v, w_v, rows_v, zbuf, acc, sem):
        c = lax.axis_index("c")
        s = lax.axis_index("s")
        wid = s * NC + c

        def zrow(i, carry):
            for t in range(nsl):
                zbuf[i, pl.ds(16 * t, 16)] = jnp.zeros((16,), jnp.float32)
            return carry

        lax.fori_loop(0, ZR, zrow, 0)
        base_r = s * RPT
        for rep in range(RPT // ZR):
            pltpu.sync_copy(zbuf, acc.at[pl.ds(base_r + rep * ZR, ZR)])
        plsc.subcore_barrier()

        ebase = wid * EPW

        def chunk(j, carry):
            off = pl.multiple_of(ebase + j * K, K)
            pltpu.sync_copy(src_hbm.at[pl.ds(off, K)], src_v)
            pltpu.sync_copy(dst_hbm.at[pl.ds(off, K)], dst_v)
            pltpu.sync_copy(wexp_hbm.at[pl.ds(off, K)], w_v)
            pltpu.async_copy(h_hbm.at[src_v], rows_v, sem).wait()

            def scale(i, cc):
                w = w_v[i]
                for t in range(nsl):
                    rows_v[i, pl.ds(16 * t, 16)] = rows_v[i, pl.ds(16 * t, 16)] * w
                return cc

            lax.fori_loop(0, K, scale, 0)
            pltpu.sync_copy(rows_v, acc.at[dst_v], add=True)
            return carry

        lax.fori_loop(0, NCHUNK, chunk, 0)
        plsc.subcore_barrier()

        pltpu.sync_copy(acc.at[pl.ds(base_r, RPT)],
                        out_hbm.at[pl.ds(c * NP + base_r, RPT)])

    return conv


_sc_conv_h = _make_sc_conv(H)
_sc_conv_c = _make_sc_conv(CP)


def _t1_body(f_ref, w0_ref, b0_ref, w1_ref, w2_ref, x0_ref, h1_ref, h2_ref):
    f = f_ref[...]
    x0_ref[...] = jnp.dot(f, w0_ref[...], preferred_element_type=jnp.float32) + b0_ref[...]
    h1_ref[...] = jnp.dot(f, w1_ref[...], preferred_element_type=jnp.float32)
    h2_ref[...] = jnp.dot(f, w2_ref[...], preferred_element_type=jnp.float32)


def _t1(features, ln1_W, ln1_b, c1_W, c2_W):
    nb = 10
    rb = N // nb
    return pl.pallas_call(
        _t1_body,
        grid=(nb,),
        in_specs=[
            pl.BlockSpec((rb, F_IN), lambda i: (i, 0)),
            pl.BlockSpec((F_IN, H), lambda i: (0, 0)),
            pl.BlockSpec((1, H), lambda i: (0, 0)),
            pl.BlockSpec((F_IN, H), lambda i: (0, 0)),
            pl.BlockSpec((F_IN, H), lambda i: (0, 0)),
        ],
        out_specs=[
            pl.BlockSpec((rb, H), lambda i: (i, 0)),
            pl.BlockSpec((rb, H), lambda i: (i, 0)),
            pl.BlockSpec((rb, H), lambda i: (i, 0)),
        ],
        out_shape=[jax.ShapeDtypeStruct((N, H), jnp.float32)] * 3,
    )(features, ln1_W, ln1_b.reshape(1, H), c1_W, c2_W)


def _t2_body(x0_ref, p1_ref, p2_ref, c1b_ref, c2b_ref, g_ref, b_ref,
             w0_ref, b0_ref, w3_ref, w4_ref, y0_ref, g3_ref, g4_ref):
    x0 = x0_ref[...]
    x1 = p1_ref[0:N, :] + p1_ref[NP:NP + N, :] + c1b_ref[...]
    x2 = p2_ref[0:N, :] + p2_ref[NP:NP + N, :] + c2b_ref[...]
    h = jnp.concatenate([x0, x1, x2], axis=1)
    m = jnp.mean(h, axis=0, keepdims=True)
    v = jnp.mean(h * h, axis=0, keepdims=True) - m * m
    hb = g_ref[...] * (h - m) * lax.rsqrt(v + 1e-5) + b_ref[...]
    y0_ref[...] = jnp.dot(hb, w0_ref[...], preferred_element_type=jnp.float32) + b0_ref[...]
    g3_ref[...] = jnp.dot(hb, w3_ref[...], preferred_element_type=jnp.float32)
    g4_ref[...] = jnp.dot(hb, w4_ref[...], preferred_element_type=jnp.float32)


def _t2(x0, p1, p2, c1_b, c2_b, bn1_g, bn1_b, ln2_W, ln2_b, c3_Wp, c4_Wp):
    return pl.pallas_call(
        _t2_body,
        out_shape=[
            jax.ShapeDtypeStruct((N, C), jnp.float32),
            jax.ShapeDtypeStruct((N, CP), jnp.float32),
            jax.ShapeDtypeStruct((N, CP), jnp.float32),
        ],
        compiler_params=pltpu.CompilerParams(vmem_limit_bytes=100 * 1024 * 1024),
    )(x0, p1, p2, c1_b.reshape(1, H), c2_b.reshape(1, H),
      bn1_g.reshape(1, 3 * H), bn1_b.reshape(1, 3 * H),
      ln2_W, ln2_b.reshape(1, C), c3_Wp, c4_Wp)


def _t3_body(y0_ref, q1_ref, q2_ref, c3b_ref, c4b_ref, g_ref, b_ref,
             w_ref, wb_ref, out_ref):
    y0 = y0_ref[...]
    y1 = q1_ref[0:N, 0:C] + q1_ref[NP:NP + N, 0:C] + c3b_ref[...]
    y2 = q2_ref[0:N, 0:C] + q2_ref[NP:NP + N, 0:C] + c4b_ref[...]
    z = jnp.concatenate([y0, y1, y2], axis=1)
    m = jnp.mean(z, axis=0, keepdims=True)
    v = jnp.mean(z * z, axis=0, keepdims=True) - m * m
    zb = g_ref[...] * (z - m) * lax.rsqrt(v + 1e-5) + b_ref[...]
    out_ref[...] = jnp.dot(zb, w_ref[...], preferred_element_type=jnp.float32) + wb_ref[...]


def _t3(y0, q1, q2, c3_b, c4_b, bn2_g, bn2_b, conv_W, conv_b):
    return pl.pallas_call(
        _t3_body,
        out_shape=jax.ShapeDtypeStruct((N, C), jnp.float32),
    )(y0, q1, q2, c3_b.reshape(1, C), c4_b.reshape(1, C),
      bn2_g.reshape(1, 3 * C), bn2_b.reshape(1, 3 * C),
      conv_W, conv_b.reshape(1, C))


def kernel(features, edge_index, edge_index2, edge_weight, edge_weight2,
           ln1_W, ln1_b, c1_W, c1_b, c2_W, c2_b, bn1_g, bn1_b,
           ln2_W, ln2_b, c3_W, c3_b, c4_W, c4_b, bn2_g, bn2_b,
           conv_W, conv_b):
    src1, dst1 = edge_index[0], edge_index[1]
    src2, dst2 = edge_index2[0], edge_index2[1]
    wexp1 = jnp.broadcast_to(edge_weight[:, None], (E, 16))
    wexp2 = jnp.broadcast_to(edge_weight2[:, None], (E, 16))

    x0, h1, h2 = _t1(features, ln1_W, ln1_b, c1_W, c2_W)
    p1 = _sc_conv_h(h1, src1, dst1, wexp1)
    p2 = _sc_conv_h(h2, src2, dst2, wexp2)

    c3_Wp = jnp.pad(c3_W, ((0, 0), (0, CP - C)))
    c4_Wp = jnp.pad(c4_W, ((0, 0), (0, CP - C)))
    y0, g3, g4 = _t2(x0, p1, p2, c1_b, c2_b, bn1_g, bn1_b,
                     ln2_W, ln2_b, c3_Wp, c4_Wp)

    q1 = _sc_conv_c(g3, src1, dst1, wexp1)
    q2 = _sc_conv_c(g4, src2, dst2, wexp2)

    return _t3(y0, q1, q2, c3_b, c4_b, bn2_g, bn2_b, conv_W, conv_b)


# speedup vs baseline: 4.1001x; 1.4175x over previous
import functools

import jax
import jax.numpy as jnp
from jax import lax
from jax.experimental import pallas as pl
from jax.experimental.pallas import tpu as pltpu
from jax.experimental.pallas import tpu_sc as plsc

N = 10000
E = 320000
F_IN = 128
H = 128
C = 40
CP = 48

NC = 2
NS = 16
NW = NC * NS
EPW = E // NW
K = 128
NCHUNK = 80
EPWP = NCHUNK * K
NP = 10240
RPT = NP // NS
ZR = 128


def _make_sc_conv(D):
    nsl = D // 16
    mesh = plsc.VectorSubcoreMesh(core_axis_name="c", subcore_axis_name="s")

    @functools.partial(
        pl.kernel,
        out_type=jax.ShapeDtypeStruct((NC * NP, D), jnp.float32),
        mesh=mesh,
        scratch_types=[
            pltpu.VMEM((3, K), jnp.int32),
            pltpu.VMEM((3, K), jnp.int32),
            pltpu.VMEM((3, K), jnp.int32),
            pltpu.VMEM((3, K), jnp.int32),
            pltpu.VMEM((K, D), jnp.float32),
            pltpu.VMEM((K, D), jnp.float32),
            pltpu.VMEM_SHARED((NP, D), jnp.float32),
            pltpu.SemaphoreType.DMA,
            pltpu.SemaphoreType.DMA,
            pltpu.SemaphoreType.DMA,
            pltpu.SemaphoreType.DMA,
            pltpu.SemaphoreType.DMA,
        ],
        compiler_params=pltpu.CompilerParams(use_tc_tiling_on_sc=False,
                                             needs_layout_passes=False),
    )
    def conv(h_hbm, slab_hbm, out_hbm,
             sb0, sb1, sb2, sb3, rows0, rows1, acc,
             gs0, gs1, ss0, ss1, slsem):
        c = lax.axis_index("c")
        s = lax.axis_index("s")
        wid = s * NC + c
        rows = (rows0, rows1)
        sbuf = (sb0, sb1, sb2, sb3)
        gsem = (gs0, gs1)
        ssem = (ss0, ss1)

        def zrow(i, carry):
            for t in range(nsl):
                rows0[i, pl.ds(16 * t, 16)] = jnp.zeros((16,), jnp.float32)
            return carry

        lax.fori_loop(0, K, zrow, 0)
        base_r = s * RPT
        for rep in range(RPT // K):
            pltpu.sync_copy(rows0, acc.at[pl.ds(base_r + rep * K, K)])

        pltpu.sync_copy(slab_hbm.at[wid * NCHUNK + 0], sb0)
        pltpu.sync_copy(slab_hbm.at[wid * NCHUNK + 1], sb1)
        pltpu.async_copy(h_hbm.at[sb0.at[0]], rows0, gs0)
        plsc.subcore_barrier()

        def do_chunk(j, q):
            b = q % 2
            nb = 1 - b
            sb = sbuf[q]
            @pl.when(j >= 1)
            def _():
                pltpu.make_async_copy(
                    rows[nb], acc.at[sbuf[(q + 3) % 4].at[1]], ssem[nb]).wait()

            @pl.when(j + 1 < NCHUNK)
            def _():
                @pl.when(j >= 1)
                def _():
                    pltpu.make_async_copy(
                        slab_hbm.at[wid * NCHUNK], sbuf[(q + 1) % 4], slsem).wait()
                pltpu.async_copy(h_hbm.at[sbuf[(q + 1) % 4].at[0]], rows[nb], gsem[nb])

            @pl.when(j + 2 < NCHUNK)
            def _():
                pltpu.async_copy(slab_hbm.at[wid * NCHUNK + j + 2],
                                 sbuf[(q + 2) % 4], slsem)

            pltpu.make_async_copy(h_hbm.at[sb.at[0]], rows[b], gsem[b]).wait()

            def scale(i, cc):
                iv = jnp.full((16,), i, jnp.int32)
                w = plsc.bitcast(
                    plsc.load_gather(sb, [jnp.full((16,), 2, jnp.int32), iv]),
                    jnp.float32)
                for t in range(nsl):
                    rows[b][i, pl.ds(16 * t, 16)] = rows[b][i, pl.ds(16 * t, 16)] * w
                return cc

            lax.fori_loop(0, K, scale, 0)
            pltpu.async_copy(rows[b], acc.at[sb.at[1]], ssem[b], add=True)

        def quad(g, carry):
            for q in range(4):
                do_chunk(4 * g + q, q)
            return carry

        lax.fori_loop(0, NCHUNK // 4, quad, 0)
        pltpu.make_async_copy(
            rows[1], acc.at[sbuf[3].at[1]], ssem[1]).wait()
        plsc.subcore_barrier()

        pltpu.sync_copy(acc.at[pl.ds(base_r, RPT)],
                        out_hbm.at[pl.ds(c * NP + base_r, RPT)])

    return conv


_sc_conv_h = _make_sc_conv(H)
_sc_conv_c = _make_sc_conv(CP)


def _pack_edges(ei, ew):
    pad = EPWP - EPW
    src = jnp.pad(ei[0].reshape(NW, EPW), ((0, 0), (0, pad)))
    dst = jnp.pad(ei[1].reshape(NW, EPW), ((0, 0), (0, pad)))
    w = jnp.pad(ew.reshape(NW, EPW), ((0, 0), (0, pad)))
    wbits = lax.bitcast_convert_type(w, jnp.int32)
    slab = jnp.stack([src.reshape(NW, NCHUNK, K),
                      dst.reshape(NW, NCHUNK, K),
                      wbits.reshape(NW, NCHUNK, K)], axis=2)
    return slab.reshape(NW * NCHUNK, 3, K)


def _t1_body(f_ref, w0_ref, b0_ref, w1_ref, w2_ref, x0_ref, h1_ref, h2_ref):
    f = f_ref[...]
    x0_ref[...] = jnp.dot(f, w0_ref[...], preferred_element_type=jnp.float32) + b0_ref[...]
    h1_ref[...] = jnp.dot(f, w1_ref[...], preferred_element_type=jnp.float32)
    h2_ref[...] = jnp.dot(f, w2_ref[...], preferred_element_type=jnp.float32)


def _t1(features, ln1_W, ln1_b, c1_W, c2_W):
    nb = 10
    rb = N // nb
    return pl.pallas_call(
        _t1_body,
        grid=(nb,),
        in_specs=[
            pl.BlockSpec((rb, F_IN), lambda i: (i, 0)),
            pl.BlockSpec((F_IN, H), lambda i: (0, 0)),
            pl.BlockSpec((1, H), lambda i: (0, 0)),
            pl.BlockSpec((F_IN, H), lambda i: (0, 0)),
            pl.BlockSpec((F_IN, H), lambda i: (0, 0)),
        ],
        out_specs=[
            pl.BlockSpec((rb, H), lambda i: (i, 0)),
            pl.BlockSpec((rb, H), lambda i: (i, 0)),
            pl.BlockSpec((rb, H), lambda i: (i, 0)),
        ],
        out_shape=[jax.ShapeDtypeStruct((N, H), jnp.float32)] * 3,
    )(features, ln1_W, ln1_b.reshape(1, H), c1_W, c2_W)


def _t2_body(x0_ref, p1_ref, p2_ref, c1b_ref, c2b_ref, g_ref, b_ref,
             w0_ref, b0_ref, w3_ref, w4_ref, y0_ref, g3_ref, g4_ref):
    x0 = x0_ref[...]
    x1 = p1_ref[0:N, :] + p1_ref[NP:NP + N, :] + c1b_ref[...]
    x2 = p2_ref[0:N, :] + p2_ref[NP:NP + N, :] + c2b_ref[...]
    h = jnp.concatenate([x0, x1, x2], axis=1)
    m = jnp.mean(h, axis=0, keepdims=True)
    v = jnp.mean(h * h, axis=0, keepdims=True) - m * m
    hb = g_ref[...] * (h - m) * lax.rsqrt(v + 1e-5) + b_ref[...]
    y0_ref[...] = jnp.dot(hb, w0_ref[...], preferred_element_type=jnp.float32) + b0_ref[...]
    g3_ref[...] = jnp.dot(hb, w3_ref[...], preferred_element_type=jnp.float32)
    g4_ref[...] = jnp.dot(hb, w4_ref[...], preferred_element_type=jnp.float32)


def _t2(x0, p1, p2, c1_b, c2_b, bn1_g, bn1_b, ln2_W, ln2_b, c3_Wp, c4_Wp):
    return pl.pallas_call(
        _t2_body,
        out_shape=[
            jax.ShapeDtypeStruct((N, C), jnp.float32),
            jax.ShapeDtypeStruct((N, CP), jnp.float32),
            jax.ShapeDtypeStruct((N, CP), jnp.float32),
        ],
        compiler_params=pltpu.CompilerParams(vmem_limit_bytes=100 * 1024 * 1024),
    )(x0, p1, p2, c1_b.reshape(1, H), c2_b.reshape(1, H),
      bn1_g.reshape(1, 3 * H), bn1_b.reshape(1, 3 * H),
      ln2_W, ln2_b.reshape(1, C), c3_Wp, c4_Wp)


def _t3_body(y0_ref, q1_ref, q2_ref, c3b_ref, c4b_ref, g_ref, b_ref,
             w_ref, wb_ref, out_ref):
    y0 = y0_ref[...]
    y1 = q1_ref[0:N, 0:C] + q1_ref[NP:NP + N, 0:C] + c3b_ref[...]
    y2 = q2_ref[0:N, 0:C] + q2_ref[NP:NP + N, 0:C] + c4b_ref[...]
    z = jnp.concatenate([y0, y1, y2], axis=1)
    m = jnp.mean(z, axis=0, keepdims=True)
    v = jnp.mean(z * z, axis=0, keepdims=True) - m * m
    zb = g_ref[...] * (z - m) * lax.rsqrt(v + 1e-5) + b_ref[...]
    out_ref[...] = jnp.dot(zb, w_ref[...], preferred_element_type=jnp.float32) + wb_ref[...]


def _t3(y0, q1, q2, c3_b, c4_b, bn2_g, bn2_b, conv_W, conv_b):
    return pl.pallas_call(
        _t3_body,
        out_shape=jax.ShapeDtypeStruct((N, C), jnp.float32),
    )(y0, q1, q2, c3_b.reshape(1, C), c4_b.reshape(1, C),
      bn2_g.reshape(1, 3 * C), bn2_b.reshape(1, 3 * C),
      conv_W, conv_b.reshape(1, C))


def kernel(features, edge_index, edge_index2, edge_weight, edge_weight2,
           ln1_W, ln1_b, c1_W, c1_b, c2_W, c2_b, bn1_g, bn1_b,
           ln2_W, ln2_b, c3_W, c3_b, c4_W, c4_b, bn2_g, bn2_b,
           conv_W, conv_b):
    slab1 = _pack_edges(edge_index, edge_weight)
    slab2 = _pack_edges(edge_index2, edge_weight2)

    x0, h1, h2 = _t1(features, ln1_W, ln1_b, c1_W, c2_W)
    p1 = _sc_conv_h(h1, slab1)
    p2 = _sc_conv_h(h2, slab2)

    c3_Wp = jnp.pad(c3_W, ((0, 0), (0, CP - C)))
    c4_Wp = jnp.pad(c4_W, ((0, 0), (0, CP - C)))
    y0, g3, g4 = _t2(x0, p1, p2, c1_b, c2_b, bn1_g, bn1_b,
                     ln2_W, ln2_b, c3_Wp, c4_Wp)

    q1 = _sc_conv_c(g3, slab1)
    q2 = _sc_conv_c(g4, slab2)

    return _t3(y0, q1, q2, c3_b, c4_b, bn2_g, bn2_b, conv_W, conv_b)
